# baseline (speedup 1.0000x reference)
import functools
import math

import numpy as np
import jax
import jax.numpy as jnp
from jax.experimental import pallas as pl
from jax.experimental.pallas import tpu as pltpu
from jax.experimental.pallas import tpu_sc as plsc

_D = 64
_S = 200
_G = 100
_T = 400
_L = 16
_NREG = _D // _L


def _pos_np():
    position = np.arange(0, _S, dtype=np.float32)[:, None]
    div_term = np.exp(
        np.arange(0, _D, 2, dtype=np.float32) * (-math.log(10000.0) / _D)
    )
    pe = np.zeros((_S, _D), dtype=np.float32)
    pe[:, 0::2] = np.sin(position * div_term)
    pe[:, 1::2] = np.cos(position * div_term)
    return pe


def _rsqrt(x):
    i = jax.lax.bitcast_convert_type(x, jnp.int32)
    y = jax.lax.bitcast_convert_type(
        jnp.int32(0x5F3759DF) - (i >> 1), jnp.float32
    )
    xh = 0.5 * x
    y = y * (1.5 - xh * y * y)
    y = y * (1.5 - xh * y * y)
    return y


@jax.jit
def _impl(idx2, table, pos, gb):
    n_tok = idx2.shape[0] * _G
    n_steps = n_tok // _T
    mesh = plsc.VectorSubcoreMesh(
        core_axis_name="core", subcore_axis_name="subcore"
    )

    @functools.partial(
        pl.kernel,
        out_type=jax.ShapeDtypeStruct((n_tok * _D,), jnp.float32),
        mesh=mesh,
        scratch_types=[
            pltpu.VMEM((_S, _D), jnp.float32),
            pltpu.VMEM((2, _D), jnp.float32),
            pltpu.VMEM((_T, _D), jnp.float32),
            pltpu.SemaphoreType.DMA,
            [pltpu.SemaphoreType.DMA] * (_T // _G),
        ],
        compiler_params=pltpu.CompilerParams(
            use_tc_tiling_on_sc=False, needs_layout_passes=False
        ),
    )
    def k(
        idx_hbm,
        table_hbm,
        pos_hbm,
        gb_hbm,
        out_hbm,
        pos_v,
        gb_v,
        rows_v,
        sem,
        gsems,
    ):
        pltpu.async_copy(gb_hbm, gb_v, sem).wait()
        pltpu.async_copy(pos_hbm, pos_v, sem).wait()

        def body(i_vmem, o_vmem):
            copies = [
                pltpu.async_copy(
                    table_hbm.at[i_vmem.at[j]],
                    rows_v.at[pl.ds(j * _G, _G)],
                    gsems[j],
                )
                for j in range(_T // _G)
            ]

            gs = [gb_v[0, pl.ds(k * _L, _L)] for k in range(_NREG)]
            bs = [gb_v[1, pl.ds(k * _L, _L)] for k in range(_NREG)]

            for j in range(_T // _G):
                copies[j].wait()
                p0 = (j * _G) % _S

                @plsc.parallel_loop(0, _G, unroll=5)
                def _(t, j=j, p0=p0):
                    row = j * _G + t
                    e = [
                        rows_v[row, pl.ds(k * _L, _L)]
                        + pos_v[p0 + t, pl.ds(k * _L, _L)]
                        for k in range(_NREG)
                    ]
                    s = (e[0] + e[1]) + (e[2] + e[3])
                    m = jnp.sum(s) * (1.0 / _D)
                    sq = (e[0] * e[0] + e[1] * e[1]) + (
                        e[2] * e[2] + e[3] * e[3]
                    )
                    var = jnp.sum(sq) * (1.0 / _D) - m * m
                    r = _rsqrt(var + 1e-5)
                    for k in range(_NREG):
                        o_vmem[pl.ds(row * _D + k * _L, _L)] = (
                            e[k] - m
                        ) * (gs[k] * r) + bs[k]

        pltpu.emit_pipeline(
            body,
            grid=(n_steps,),
            in_specs=[
                pl.BlockSpec((_T // _G, _G), lambda i: (i, 0)),
            ],
            out_specs=[pl.BlockSpec((_T * _D,), lambda i: (i,))],
            core_axis_name=("core", "subcore"),
            dimension_semantics=(pltpu.PARALLEL,),
            trace_scopes=False,
        )(idx_hbm, out_hbm)

    return k(idx2, table, pos, gb)


def kernel(token_ids, token_table, ln_gamma, ln_beta):
    B, S = token_ids.shape
    idx2 = token_ids.reshape(-1, _G).astype(jnp.int32)
    pos = jnp.asarray(_pos_np())
    gb = jnp.stack([ln_gamma, ln_beta])
    return _impl(idx2, token_table, pos, gb).reshape(B, S, _D)


# speedup vs baseline: 1.1143x; 1.1143x over previous
import functools
import math

import numpy as np
import jax
import jax.numpy as jnp
from jax.experimental import pallas as pl
from jax.experimental.pallas import tpu as pltpu
from jax.experimental.pallas import tpu_sc as plsc

_D = 64
_S = 200
_G = 100
_T = 400
_L = 16
_NREG = _D // _L
_ROWS_TC = 32


def _pos_np():
    position = np.arange(0, _S, dtype=np.float32)[:, None]
    div_term = np.exp(
        np.arange(0, _D, 2, dtype=np.float32) * (-math.log(10000.0) / _D)
    )
    pe = np.zeros((_S, _D), dtype=np.float32)
    pe[:, 0::2] = np.sin(position * div_term)
    pe[:, 1::2] = np.cos(position * div_term)
    return pe


def _rsqrt(x):
    i = jax.lax.bitcast_convert_type(x, jnp.int32)
    y = jax.lax.bitcast_convert_type(
        jnp.int32(0x5F3759DF) - (i >> 1), jnp.float32
    )
    xh = 0.5 * x
    y = y * (1.5 - xh * y * y)
    y = y * (1.5 - xh * y * y)
    return y


def _sc_part(idx2, table, pos, gb):
    n_tok = idx2.shape[0] * _G
    n_steps = n_tok // _T
    mesh = plsc.VectorSubcoreMesh(
        core_axis_name="core", subcore_axis_name="subcore"
    )

    @functools.partial(
        pl.kernel,
        out_type=jax.ShapeDtypeStruct((n_tok // 2, 2 * _D), jnp.float32),
        mesh=mesh,
        scratch_types=[
            pltpu.VMEM((_S, _D), jnp.float32),
            pltpu.VMEM((2, _D), jnp.float32),
            pltpu.VMEM((_T, _D), jnp.float32),
            pltpu.SemaphoreType.DMA,
            [pltpu.SemaphoreType.DMA] * (_T // _G),
        ],
        compiler_params=pltpu.CompilerParams(
            use_tc_tiling_on_sc=False, needs_layout_passes=False
        ),
    )
    def k(
        idx_hbm,
        table_hbm,
        pos_hbm,
        gb_hbm,
        out_hbm,
        pos_v,
        gb_v,
        rows_v,
        sem,
        gsems,
    ):
        pltpu.async_copy(gb_hbm, gb_v, sem).wait()
        pltpu.async_copy(pos_hbm, pos_v, sem).wait()

        def body(i_vmem, o_vmem):
            copies = [
                pltpu.async_copy(
                    table_hbm.at[i_vmem.at[j]],
                    rows_v.at[pl.ds(j * _G, _G)],
                    gsems[j],
                )
                for j in range(_T // _G)
            ]

            gs = [gb_v[0, pl.ds(k * _L, _L)] for k in range(_NREG)]
            bs = [gb_v[1, pl.ds(k * _L, _L)] for k in range(_NREG)]

            for j in range(_T // _G):
                copies[j].wait()
                p0 = (j * _G) % _S

                @plsc.parallel_loop(0, _G, unroll=5)
                def _(t, j=j, p0=p0):
                    row = j * _G + t
                    e = [
                        rows_v[row, pl.ds(k * _L, _L)]
                        + pos_v[p0 + t, pl.ds(k * _L, _L)]
                        for k in range(_NREG)
                    ]
                    s = (e[0] + e[1]) + (e[2] + e[3])
                    m = jnp.sum(s) * (1.0 / _D)
                    sq = (e[0] * e[0] + e[1] * e[1]) + (
                        e[2] * e[2] + e[3] * e[3]
                    )
                    var = jnp.sum(sq) * (1.0 / _D) - m * m
                    r = _rsqrt(var + 1e-5)
                    for k in range(_NREG):
                        o_vmem[row, pl.ds(k * _L, _L)] = (e[k] - m) * (
                            gs[k] * r
                        ) + bs[k]

        pltpu.emit_pipeline(
            body,
            grid=(n_steps,),
            in_specs=[
                pl.BlockSpec((_T // _G, _G), lambda i: (i, 0)),
            ],
            out_specs=[
                pl.BlockSpec(
                    (_T, _D),
                    lambda i: ((i // 16) * 8 + i % 8, (i % 16) // 8),
                )
            ],
            core_axis_name=("core", "subcore"),
            dimension_semantics=(pltpu.PARALLEL,),
            trace_scopes=False,
        )(idx_hbm, out_hbm)

    return k(idx2, table, pos, gb)


_TC_TOK = 6400


def _tc_epilogue(c2, n_tok):
    half = _TC_TOK // 2

    def body(c_ref, o_ref):
        x = c_ref[...]
        o_ref[pl.ds(0, half), :] = x[:, 0:_D]
        o_ref[pl.ds(half, half), :] = x[:, _D:2 * _D]

    return pl.pallas_call(
        body,
        grid=(n_tok // _TC_TOK,),
        in_specs=[pl.BlockSpec((half, 2 * _D), lambda i: (i, 0))],
        out_specs=pl.BlockSpec((_TC_TOK, _D), lambda i: (i, 0)),
        out_shape=jax.ShapeDtypeStruct((n_tok, _D), jnp.float32),
    )(c2)


@jax.jit
def _impl(idx2, table, pos, gb):
    n_tok = idx2.shape[0] * _G
    c2 = _sc_part(idx2, table, pos, gb)
    return _tc_epilogue(c2, n_tok)


def kernel(token_ids, token_table, ln_gamma, ln_beta):
    B, S = token_ids.shape
    idx2 = token_ids.reshape(-1, _G).astype(jnp.int32)
    pos = jnp.asarray(_pos_np())
    gb = jnp.stack([ln_gamma, ln_beta])
    return _impl(idx2, token_table, pos, gb).reshape(B, S, _D)
